# Initial kernel scaffold; baseline (speedup 1.0000x reference)
import jax
import jax.numpy as jnp
from jax.experimental import pallas as pl


def kernel(x, edge_index, batch_size, W_self, W_neigh, bias, gamma, beta):
    raise NotImplementedError("write your pallas kernel here")


# speedup vs baseline: 10.0744x; 10.0744x over previous
import functools

import jax
import jax.numpy as jnp
from jax import lax
from jax.experimental import pallas as pl
from jax.experimental.pallas import tpu as pltpu
from jax.experimental.pallas import tpu_sc as plsc

N = 5000
D = 128
E = 32768
BATCH = 16

NC = 2
NS = 16
BPC = BATCH // NC

ROWS_PER_TILE = 320
NPAD = ROWS_PER_TILE * NS
EPT = E // NS
CHUNK = 128
NCHUNKS = EPT // CHUNK

_sc_mesh = plsc.VectorSubcoreMesh(core_axis_name="c", subcore_axis_name="s")


@functools.partial(
    pl.kernel,
    out_type=[
        jax.ShapeDtypeStruct((BATCH, NPAD, D), jnp.float32),
        jax.ShapeDtypeStruct((NPAD, D), jnp.float32),
    ],
    mesh=_sc_mesh,
    scratch_types=[
        pltpu.VMEM((NCHUNKS, CHUNK), jnp.int32),
        pltpu.VMEM((NCHUNKS, CHUNK), jnp.int32),
        pltpu.VMEM((CHUNK, D), jnp.float32),
        pltpu.VMEM((ROWS_PER_TILE, D), jnp.float32),
        pltpu.VMEM_SHARED((NPAD, D), jnp.float32),
        pltpu.SemaphoreType.DMA,
    ],
)
def _sc_scatter(x_hbm, src_hbm, dst_hbm, agg_hbm, deg_hbm,
                srcv, dstv, rows, zbuf, agg_sh, sem):
    c = lax.axis_index("c")
    s = lax.axis_index("s")
    my = pl.ds(s * ROWS_PER_TILE, ROWS_PER_TILE)

    zero16 = jnp.zeros((16,), jnp.float32)

    def _zrow(i, _):
        for l in range(D // 16):
            zbuf[i, pl.ds(l * 16, 16)] = zero16
        return 0

    lax.fori_loop(0, ROWS_PER_TILE, _zrow, 0)

    pltpu.sync_copy(src_hbm.at[pl.ds(s * NCHUNKS, NCHUNKS)], srcv)
    pltpu.sync_copy(dst_hbm.at[pl.ds(s * NCHUNKS, NCHUNKS)], dstv)

    @pl.when(c == 0)
    def _deg():
        one16 = jnp.full((16,), 1.0, jnp.float32)

        def _orow(i, _):
            for l in range(D // 16):
                rows[i, pl.ds(l * 16, 16)] = one16
            return 0

        lax.fori_loop(0, CHUNK, _orow, 0)
        pltpu.sync_copy(zbuf, agg_sh.at[my])
        plsc.subcore_barrier()
        for j in range(NCHUNKS):
            pltpu.sync_copy(rows, agg_sh.at[dstv.at[j]], add=True)
        plsc.subcore_barrier()
        pltpu.sync_copy(agg_sh.at[my], deg_hbm.at[my])

    base0 = c * (BPC * N)

    def _shift(i, _):
        for l in range(CHUNK // 16):
            sl = pl.ds(l * 16, 16)
            srcv[i, sl] = srcv[i, sl] + base0
        return 0

    lax.fori_loop(0, NCHUNKS, _shift, 0)

    def _batch(b, _):
        pltpu.sync_copy(zbuf, agg_sh.at[my])
        plsc.subcore_barrier()
        for j in range(NCHUNKS):
            pltpu.async_copy(x_hbm.at[srcv.at[j]], rows, sem).wait()
            pltpu.sync_copy(rows, agg_sh.at[dstv.at[j]], add=True)
        plsc.subcore_barrier()
        bg = c * BPC + b
        pltpu.sync_copy(agg_sh.at[my], agg_hbm.at[bg, my])

        def _bump(i, _):
            for l in range(CHUNK // 16):
                sl = pl.ds(l * 16, 16)
                srcv[i, sl] = srcv[i, sl] + N
            return 0

        lax.fori_loop(0, NCHUNKS, _bump, 0)
        return 0

    lax.fori_loop(0, BPC, _batch, 0)


BN = 1000


def _dense_body(x_ref, agg_ref, deg_ref, ws_ref, wn_ref, b_ref, g_ref, be_ref,
                o_ref):
    xb = x_ref[0]
    inv = 1.0 / jnp.maximum(deg_ref[...], 1.0)
    neigh = agg_ref[0] * inv
    out = (jnp.dot(xb, ws_ref[...], preferred_element_type=jnp.float32)
           + jnp.dot(neigh, wn_ref[...], preferred_element_type=jnp.float32)
           + b_ref[...])
    mu = jnp.mean(out, axis=-1, keepdims=True)
    var = jnp.mean((out - mu) ** 2, axis=-1, keepdims=True)
    out = (out - mu) * lax.rsqrt(var + 1e-5) * g_ref[...] + be_ref[...]
    o_ref[0] = jnp.maximum(out, 0.0)


_dense = pl.pallas_call(
    _dense_body,
    grid=(BATCH, N // BN),
    in_specs=[
        pl.BlockSpec((1, BN, D), lambda b, j: (b, j, 0)),
        pl.BlockSpec((1, BN, D), lambda b, j: (b, j, 0)),
        pl.BlockSpec((BN, D), lambda b, j: (j, 0)),
        pl.BlockSpec((D, D), lambda b, j: (0, 0)),
        pl.BlockSpec((D, D), lambda b, j: (0, 0)),
        pl.BlockSpec((1, D), lambda b, j: (0, 0)),
        pl.BlockSpec((1, D), lambda b, j: (0, 0)),
        pl.BlockSpec((1, D), lambda b, j: (0, 0)),
    ],
    out_specs=pl.BlockSpec((1, BN, D), lambda b, j: (b, j, 0)),
    out_shape=jax.ShapeDtypeStruct((BATCH, N, D), jnp.float32),
    compiler_params=pltpu.CompilerParams(
        dimension_semantics=("parallel", "parallel")),
)


def kernel(x, edge_index, batch_size, W_self, W_neigh, bias, gamma, beta):
    x_flat = x.reshape(BATCH * N, D)
    src2d = edge_index[0].reshape(E // CHUNK, CHUNK)
    dst2d = edge_index[1].reshape(E // CHUNK, CHUNK)
    agg_pad, deg_pad = _sc_scatter(x_flat, src2d, dst2d)
    out = _dense(x, agg_pad, deg_pad, W_self, W_neigh,
                 bias.reshape(1, D), gamma.reshape(1, D), beta.reshape(1, D))
    return out


# speedup vs baseline: 13.2463x; 1.3148x over previous
import functools

import jax
import jax.numpy as jnp
from jax import lax
from jax.experimental import pallas as pl
from jax.experimental.pallas import tpu as pltpu
from jax.experimental.pallas import tpu_sc as plsc

N = 5000
D = 128
E = 32768
BATCH = 16

NC = 2
NS = 16
BPC = BATCH // NC

ROWS_PER_TILE = 320
NPAD = ROWS_PER_TILE * NS
EPT = E // NS
CHUNK = 128
NCHUNKS = EPT // CHUNK

_sc_mesh = plsc.VectorSubcoreMesh(core_axis_name="c", subcore_axis_name="s")


@functools.partial(
    pl.kernel,
    out_type=[
        jax.ShapeDtypeStruct((BATCH, NPAD, D), jnp.float32),
        jax.ShapeDtypeStruct((NPAD, D), jnp.float32),
    ],
    mesh=_sc_mesh,
    scratch_types=[
        pltpu.VMEM((NCHUNKS, CHUNK), jnp.int32),
        pltpu.VMEM((NCHUNKS, CHUNK), jnp.int32),
        pltpu.VMEM((2, CHUNK, D), jnp.float32),
        pltpu.VMEM((ROWS_PER_TILE, D), jnp.float32),
        pltpu.VMEM_SHARED((NPAD, D), jnp.float32),
        pltpu.SemaphoreType.DMA,
        pltpu.SemaphoreType.DMA,
    ],
)
def _sc_scatter(x_hbm, src_hbm, dst_hbm, agg_hbm, deg_hbm,
                srcv, dstv, rows, zbuf, agg_sh, sem0, sem1):
    c = lax.axis_index("c")
    s = lax.axis_index("s")
    my = pl.ds(s * ROWS_PER_TILE, ROWS_PER_TILE)

    zero16 = jnp.zeros((16,), jnp.float32)

    def _zrow(i, _):
        for l in range(D // 16):
            zbuf[i, pl.ds(l * 16, 16)] = zero16
        return 0

    lax.fori_loop(0, ROWS_PER_TILE, _zrow, 0)

    pltpu.sync_copy(src_hbm.at[pl.ds(s * NCHUNKS, NCHUNKS)], srcv)
    pltpu.sync_copy(dst_hbm.at[pl.ds(s * NCHUNKS, NCHUNKS)], dstv)

    @pl.when(c == 0)
    def _deg():
        one16 = jnp.full((16,), 1.0, jnp.float32)

        def _orow(i, _):
            for l in range(D // 16):
                rows[0, i, pl.ds(l * 16, 16)] = one16
            return 0

        lax.fori_loop(0, CHUNK, _orow, 0)
        pltpu.sync_copy(zbuf, agg_sh.at[my])
        plsc.subcore_barrier()
        for j in range(NCHUNKS):
            pltpu.sync_copy(rows.at[0], agg_sh.at[dstv.at[j]], add=True)
        plsc.subcore_barrier()
        pltpu.sync_copy(agg_sh.at[my], deg_hbm.at[my])

    base0 = c * (BPC * N)

    def _shift(i, _):
        for l in range(CHUNK // 16):
            sl = pl.ds(l * 16, 16)
            srcv[i, sl] = srcv[i, sl] + base0
        return 0

    lax.fori_loop(0, NCHUNKS, _shift, 0)

    def _batch(b, _):
        pltpu.sync_copy(zbuf, agg_sh.at[my])
        plsc.subcore_barrier()
        sems = (sem0, sem1)
        pend = pltpu.async_copy(x_hbm.at[srcv.at[0]], rows.at[0], sems[0])
        for j in range(NCHUNKS):
            if j + 1 < NCHUNKS:
                nxt = pltpu.async_copy(
                    x_hbm.at[srcv.at[j + 1]], rows.at[(j + 1) % 2],
                    sems[(j + 1) % 2])
            pend.wait()
            pltpu.sync_copy(rows.at[j % 2], agg_sh.at[dstv.at[j]], add=True)
            if j + 1 < NCHUNKS:
                pend = nxt
        plsc.subcore_barrier()
        bg = c * BPC + b
        pltpu.sync_copy(agg_sh.at[my], agg_hbm.at[bg, my])

        def _bump(i, _):
            for l in range(CHUNK // 16):
                sl = pl.ds(l * 16, 16)
                srcv[i, sl] = srcv[i, sl] + N
            return 0

        lax.fori_loop(0, NCHUNKS, _bump, 0)
        return 0

    lax.fori_loop(0, BPC, _batch, 0)


BN = 1000


def _dense_body(x_ref, agg_ref, deg_ref, ws_ref, wn_ref, b_ref, g_ref, be_ref,
                o_ref):
    xb = x_ref[0]
    inv = 1.0 / jnp.maximum(deg_ref[...], 1.0)
    neigh = agg_ref[0] * inv
    out = (jnp.dot(xb, ws_ref[...], preferred_element_type=jnp.float32)
           + jnp.dot(neigh, wn_ref[...], preferred_element_type=jnp.float32)
           + b_ref[...])
    mu = jnp.mean(out, axis=-1, keepdims=True)
    var = jnp.mean((out - mu) ** 2, axis=-1, keepdims=True)
    out = (out - mu) * lax.rsqrt(var + 1e-5) * g_ref[...] + be_ref[...]
    o_ref[0] = jnp.maximum(out, 0.0)


_dense = pl.pallas_call(
    _dense_body,
    grid=(BATCH, N // BN),
    in_specs=[
        pl.BlockSpec((1, BN, D), lambda b, j: (b, j, 0)),
        pl.BlockSpec((1, BN, D), lambda b, j: (b, j, 0)),
        pl.BlockSpec((BN, D), lambda b, j: (j, 0)),
        pl.BlockSpec((D, D), lambda b, j: (0, 0)),
        pl.BlockSpec((D, D), lambda b, j: (0, 0)),
        pl.BlockSpec((1, D), lambda b, j: (0, 0)),
        pl.BlockSpec((1, D), lambda b, j: (0, 0)),
        pl.BlockSpec((1, D), lambda b, j: (0, 0)),
    ],
    out_specs=pl.BlockSpec((1, BN, D), lambda b, j: (b, j, 0)),
    out_shape=jax.ShapeDtypeStruct((BATCH, N, D), jnp.float32),
    compiler_params=pltpu.CompilerParams(
        dimension_semantics=("parallel", "parallel")),
)


def kernel(x, edge_index, batch_size, W_self, W_neigh, bias, gamma, beta):
    x_flat = x.reshape(BATCH * N, D)
    src2d = edge_index[0].reshape(E // CHUNK, CHUNK)
    dst2d = edge_index[1].reshape(E // CHUNK, CHUNK)
    agg_pad, deg_pad = _sc_scatter(x_flat, src2d, dst2d)
    out = _dense(x, agg_pad, deg_pad, W_self, W_neigh,
                 bias.reshape(1, D), gamma.reshape(1, D), beta.reshape(1, D))
    return out


# speedup vs baseline: 13.8617x; 1.0465x over previous
import functools

import jax
import jax.numpy as jnp
from jax import lax
from jax.experimental import pallas as pl
from jax.experimental.pallas import tpu as pltpu
from jax.experimental.pallas import tpu_sc as plsc

N = 5000
D = 128
E = 32768
BATCH = 16

NC = 2
NS = 16
BPC = BATCH // NC

ROWS_PER_TILE = 320
NPAD = ROWS_PER_TILE * NS
EPT = E // NS
CHUNK = 128
NCHUNKS = EPT // CHUNK

_sc_mesh = plsc.VectorSubcoreMesh(core_axis_name="c", subcore_axis_name="s")


@functools.partial(
    pl.kernel,
    out_type=[
        jax.ShapeDtypeStruct((BATCH, NPAD, D), jnp.float32),
        jax.ShapeDtypeStruct((NPAD, D), jnp.float32),
    ],
    mesh=_sc_mesh,
    scratch_types=[
        pltpu.VMEM((NCHUNKS, CHUNK), jnp.int32),
        pltpu.VMEM((NCHUNKS, CHUNK), jnp.int32),
        pltpu.VMEM((3, CHUNK, D), jnp.float32),
        pltpu.VMEM((ROWS_PER_TILE // 5, D), jnp.float32),
        pltpu.VMEM_SHARED((NPAD, D), jnp.float32),
        pltpu.SemaphoreType.DMA,
        pltpu.SemaphoreType.DMA,
        pltpu.SemaphoreType.DMA,
        pltpu.SemaphoreType.DMA,
    ],
)
def _sc_scatter(x_hbm, src_hbm, dst_hbm, agg_hbm, deg_hbm,
                srcv, dstv, rows, zbuf, agg_sh, sem0, sem1, sem2, sem3):
    c = lax.axis_index("c")
    s = lax.axis_index("s")
    my = pl.ds(s * ROWS_PER_TILE, ROWS_PER_TILE)

    zero16 = jnp.zeros((16,), jnp.float32)

    def _zrow(i, _):
        for l in range(D // 16):
            zbuf[i, pl.ds(l * 16, 16)] = zero16
        return 0

    lax.fori_loop(0, ROWS_PER_TILE // 5, _zrow, 0)

    def _zero_my_slice():
        for z in range(5):
            pltpu.sync_copy(
                zbuf,
                agg_sh.at[pl.ds(s * ROWS_PER_TILE + z * (ROWS_PER_TILE // 5),
                                ROWS_PER_TILE // 5)])

    pltpu.sync_copy(src_hbm.at[pl.ds(s * NCHUNKS, NCHUNKS)], srcv)
    pltpu.sync_copy(dst_hbm.at[pl.ds(s * NCHUNKS, NCHUNKS)], dstv)

    @pl.when(c == 0)
    def _deg():
        one16 = jnp.full((16,), 1.0, jnp.float32)

        def _orow(i, _):
            for l in range(D // 16):
                rows[0, i, pl.ds(l * 16, 16)] = one16
            return 0

        lax.fori_loop(0, CHUNK, _orow, 0)
        _zero_my_slice()
        plsc.subcore_barrier()
        for j in range(NCHUNKS):
            pltpu.sync_copy(rows.at[0], agg_sh.at[dstv.at[j]], add=True)
        plsc.subcore_barrier()
        pltpu.sync_copy(agg_sh.at[my], deg_hbm.at[my])

    base0 = c * (BPC * N)

    def _shift(i, _):
        for l in range(CHUNK // 16):
            sl = pl.ds(l * 16, 16)
            srcv[i, sl] = srcv[i, sl] + base0
        return 0

    lax.fori_loop(0, NCHUNKS, _shift, 0)

    NBUF = 3
    sems = (sem0, sem1, sem2, sem3)

    def _batch(b, _):
        _zero_my_slice()
        plsc.subcore_barrier()
        gat = [None] * NCHUNKS
        scat = [None] * NCHUNKS
        for j in range(NBUF - 1):
            gat[j] = pltpu.async_copy(
                x_hbm.at[srcv.at[j]], rows.at[j % NBUF], sems[j % NBUF])
        for j in range(NCHUNKS):
            if j - 1 >= 0:
                scat[j - 1].wait()
            if j + NBUF - 1 < NCHUNKS:
                jn = j + NBUF - 1
                gat[jn] = pltpu.async_copy(
                    x_hbm.at[srcv.at[jn]], rows.at[jn % NBUF], sems[jn % NBUF])
            gat[j].wait()
            scat[j] = pltpu.async_copy(
                rows.at[j % NBUF], agg_sh.at[dstv.at[j]], sems[j % NBUF],
                add=True)
        scat[NCHUNKS - 1].wait()
        plsc.subcore_barrier()
        bg = c * BPC + b
        pltpu.sync_copy(agg_sh.at[my], agg_hbm.at[bg, my])

        def _bump(i, _):
            for l in range(CHUNK // 16):
                sl = pl.ds(l * 16, 16)
                srcv[i, sl] = srcv[i, sl] + N
            return 0

        lax.fori_loop(0, NCHUNKS, _bump, 0)
        return 0

    lax.fori_loop(0, BPC, _batch, 0)


BN = 1000


def _dense_body(x_ref, agg_ref, deg_ref, ws_ref, wn_ref, b_ref, g_ref, be_ref,
                o_ref):
    xb = x_ref[0]
    inv = 1.0 / jnp.maximum(deg_ref[...], 1.0)
    neigh = agg_ref[0] * inv
    out = (jnp.dot(xb, ws_ref[...], preferred_element_type=jnp.float32)
           + jnp.dot(neigh, wn_ref[...], preferred_element_type=jnp.float32)
           + b_ref[...])
    mu = jnp.mean(out, axis=-1, keepdims=True)
    var = jnp.mean((out - mu) ** 2, axis=-1, keepdims=True)
    out = (out - mu) * lax.rsqrt(var + 1e-5) * g_ref[...] + be_ref[...]
    o_ref[0] = jnp.maximum(out, 0.0)


_dense = pl.pallas_call(
    _dense_body,
    grid=(BATCH, N // BN),
    in_specs=[
        pl.BlockSpec((1, BN, D), lambda b, j: (b, j, 0)),
        pl.BlockSpec((1, BN, D), lambda b, j: (b, j, 0)),
        pl.BlockSpec((BN, D), lambda b, j: (j, 0)),
        pl.BlockSpec((D, D), lambda b, j: (0, 0)),
        pl.BlockSpec((D, D), lambda b, j: (0, 0)),
        pl.BlockSpec((1, D), lambda b, j: (0, 0)),
        pl.BlockSpec((1, D), lambda b, j: (0, 0)),
        pl.BlockSpec((1, D), lambda b, j: (0, 0)),
    ],
    out_specs=pl.BlockSpec((1, BN, D), lambda b, j: (b, j, 0)),
    out_shape=jax.ShapeDtypeStruct((BATCH, N, D), jnp.float32),
    compiler_params=pltpu.CompilerParams(
        dimension_semantics=("parallel", "parallel")),
)


def kernel(x, edge_index, batch_size, W_self, W_neigh, bias, gamma, beta):
    x_flat = x.reshape(BATCH * N, D)
    src2d = edge_index[0].reshape(E // CHUNK, CHUNK)
    dst2d = edge_index[1].reshape(E // CHUNK, CHUNK)
    agg_pad, deg_pad = _sc_scatter(x_flat, src2d, dst2d)
    out = _dense(x, agg_pad, deg_pad, W_self, W_neigh,
                 bias.reshape(1, D), gamma.reshape(1, D), beta.reshape(1, D))
    return out
